# baseline (speedup 1.0000x reference)
import functools
import jax
import jax.numpy as jnp
from jax import lax
from jax.experimental import pallas as pl
from jax.experimental.pallas import tpu as pltpu, tpu_sc as plsc

_C = 32000
_PAD = 0
_SM = 0.1
_CONF = 1.0 - _SM
_EPS = _SM / (_C - 2)
_BR = 128
_LW = 128
_NCH = _C // _LW

_info = plsc.get_sparse_core_info()
_NC, _NS, _L = _info.num_cores, _info.num_subcores, _info.num_lanes
_NW = _NC * _NS
_BATCH = 32


def _sc_gather(pred_hbm, tgt_hbm, out_hbm, t_v, rows_v, comp_v, sem):
    n_per_w = tgt_hbm.shape[0] // _NW
    wid = lax.axis_index("s") * _NC + lax.axis_index("c")
    base = wid * n_per_w
    pltpu.sync_copy(tgt_hbm.at[pl.ds(base, n_per_w)], t_v)
    for b in range(n_per_w // _BATCH):
        copies = []
        for k in range(_BATCH):
            j = b * _BATCH + k
            t = t_v[pl.ds((j // _L) * _L, _L)][j % _L]
            ct = pl.multiple_of(jnp.bitwise_and(t, -128), 128)
            r0 = pl.multiple_of(base + (j // 8) * 8, 8)
            copies.append(
                pltpu.async_copy(
                    pred_hbm.at[pl.ds(r0, 8), pl.ds(ct, 128)],
                    rows_v.at[k],
                    sem,
                )
            )
        for cp in copies:
            cp.wait()
        for k in range(_BATCH):
            j = b * _BATCH + k
            for g in range(_LW // _L):
                comp_v[pl.ds(j * _LW + g * _L, _L)] = (
                    rows_v[k, j % 8, pl.ds(g * _L, _L)])
    pltpu.sync_copy(comp_v, out_hbm.at[pl.ds(base * _LW, n_per_w * _LW)])


def _gather_windows(pred, target):
    n = target.shape[0]
    n_per_w = n // _NW
    mesh = plsc.VectorSubcoreMesh(core_axis_name="c", subcore_axis_name="s")
    f = functools.partial(
        pl.kernel,
        mesh=mesh,
        out_type=jax.ShapeDtypeStruct((n * _LW,), jnp.float32),
        scratch_types=[
            pltpu.VMEM((n_per_w,), jnp.int32),
            pltpu.VMEM((_BATCH, 8, _LW), jnp.float32),
            pltpu.VMEM((n_per_w * _LW,), jnp.float32),
            pltpu.SemaphoreType.DMA,
        ],
    )(_sc_gather)
    return f(pred, target)


_RW = 48
_NSC = _RW * _NW
_CCH = 3200
_NCC = _C // _CCH
_VPC = _CCH // _L


def _sc_dense(pred_hbm, m_hbm, se_hbm, sx_hbm, p0_hbm,
              buf, mo, seo, sxo, p0o, tb8):
    wid = lax.axis_index("s") * _NC + lax.axis_index("c")
    row0w = wid * _RW
    zer = jnp.zeros((_L,), jnp.float32)
    for g in range(_RW // 8):
        r0 = pl.multiple_of(row0w + g * 8, 8)

        def chunk(cc, carry):
            cc128 = pl.multiple_of(cc * _CCH, 128)
            pltpu.sync_copy(pred_hbm.at[pl.ds(r0, 8), pl.ds(cc128, _CCH)],
                            buf)
            out = []
            for k in range(8):
                macc, seacc, sxacc = carry[3 * k:3 * k + 3]

                def pa(v, c):
                    x = buf[k, pl.ds(v * _L, _L)]
                    return (jnp.maximum(c[0], x), c[1] + x)

                cm, csx = lax.fori_loop(
                    0, _VPC, pa,
                    (jnp.full((_L,), -jnp.inf, jnp.float32), zer))
                mnew = jnp.maximum(macc, cm)
                scale = jnp.exp(macc - mnew)

                def pb(v, s):
                    return s + jnp.exp(buf[k, pl.ds(v * _L, _L)] - mnew)

                cse = lax.fori_loop(0, _VPC, pb, zer)
                out += [mnew, seacc * scale + cse, sxacc + csx]
            return tuple(out)

        ninf = jnp.full((_L,), -jnp.inf, jnp.float32)
        init = tuple([ninf, zer, zer] * 8)
        fin = lax.fori_loop(0, _NCC, chunk, init)
        for k in range(8):
            macc, seacc, sxacc = fin[3 * k:3 * k + 3]
            o16 = (g * 8 + k) * _L
            mo[pl.ds(o16, _L)] = macc
            seo[pl.ds(o16, _L)] = seacc
            sxo[pl.ds(o16, _L)] = sxacc
    for g in range(_RW // 8):
        r0 = pl.multiple_of(row0w + g * 8, 8)
        pltpu.sync_copy(pred_hbm.at[pl.ds(r0, 8), pl.ds(0, _LW)], tb8)
        for k in range(8):
            p0o[pl.ds((g * 8 + k) * _L, _L)] = tb8[k, pl.ds(0, _L)]
    pltpu.sync_copy(mo, m_hbm.at[pl.ds(row0w * _L, _RW * _L)])
    pltpu.sync_copy(seo, se_hbm.at[pl.ds(row0w * _L, _RW * _L)])
    pltpu.sync_copy(sxo, sx_hbm.at[pl.ds(row0w * _L, _RW * _L)])
    pltpu.sync_copy(p0o, p0_hbm.at[pl.ds(row0w * _L, _RW * _L)])


def _sc_dense_stats(pred):
    mesh = plsc.VectorSubcoreMesh(core_axis_name="c", subcore_axis_name="s")
    o = jax.ShapeDtypeStruct((_NSC * _L,), jnp.float32)
    f = functools.partial(
        pl.kernel,
        mesh=mesh,
        out_type=[o, o, o, o],
        scratch_types=[
            pltpu.VMEM((8, _CCH), jnp.float32),
            pltpu.VMEM((_RW * _L,), jnp.float32),
            pltpu.VMEM((_RW * _L,), jnp.float32),
            pltpu.VMEM((_RW * _L,), jnp.float32),
            pltpu.VMEM((_RW * _L,), jnp.float32),
            pltpu.VMEM((8, _LW), jnp.float32),
        ],
    )(_sc_dense)
    return f(pred)


def _body_dense(x_ref, m_ref, se_ref, sx_ref, p0_ref):
    lane = jax.lax.broadcasted_iota(jnp.int32, (_BR, _LW), 1)

    m = x_ref[:, 0:_LW]
    sp = m
    for c in range(1, _NCH):
        x = x_ref[:, c * _LW:(c + 1) * _LW]
        m = jnp.maximum(m, x)
        sp = sp + x

    mb = jnp.max(m, axis=1, keepdims=True)

    s = jnp.exp(x_ref[:, 0:_LW] - mb)
    for c in range(1, _NCH):
        s = s + jnp.exp(x_ref[:, c * _LW:(c + 1) * _LW] - mb)

    m_ref[...] = mb
    se_ref[...] = jnp.sum(s, axis=1, keepdims=True)
    sx_ref[...] = jnp.sum(sp, axis=1, keepdims=True)
    x0 = x_ref[:, 0:_LW]
    p0_ref[...] = jnp.sum(jnp.where(lane == 0, x0, 0.0), axis=1, keepdims=True)


def _body_combine(t_ref, w_ref, m16_ref, se16_ref, sx16_ref, p016_ref,
                  mt_ref, set_ref, sxt_ref, p0t_ref, o_ref):
    n = t_ref.shape[0]
    tb = t_ref[...]
    lane = jax.lax.broadcasted_iota(jnp.int32, (n, _LW), 1)
    m16 = m16_ref[...]
    mb_sc = jnp.max(m16, axis=1, keepdims=True)
    se_sc = jnp.sum(se16_ref[...] * jnp.exp(m16 - mb_sc),
                    axis=1, keepdims=True)
    z_sc = mb_sc + jnp.log(se_sc)
    sx_sc = jnp.sum(sx16_ref[...], axis=1, keepdims=True)
    p0_sc = p016_ref[:, 0:1]
    z_tc = mt_ref[...] + jnp.log(set_ref[...])
    z = jnp.concatenate([z_sc, z_tc], axis=0)
    spr = jnp.concatenate([sx_sc, sxt_ref[...]], axis=0)
    p0 = jnp.concatenate([p0_sc, p0t_ref[...]], axis=0)
    tlane = jnp.bitwise_and(tb, 127)
    ptr = jnp.sum(jnp.where(lane == tlane, w_ref[...], 0.0),
                  axis=1, keepdims=True)
    lt = ptr - z
    l0 = p0 - z
    srow = spr - _C * z
    loss = -(_CONF * lt + _EPS * (srow - l0 - lt))
    o_ref[...] = jnp.where(tb == _PAD, 0.0, loss)


def kernel(pred, target):
    n = pred.shape[0]
    nb_tc = (n - _NSC) // _BR
    off = _NSC // _BR
    t32 = target.astype(jnp.int32)
    win = _gather_windows(pred, t32).reshape(n, _LW)
    ms, ses, sxs, p0s = _sc_dense_stats(pred)
    t2 = t32.reshape(n, 1)
    stat = jax.ShapeDtypeStruct((n - _NSC, 1), jnp.float32)
    mt, set_, sxt, p0t = pl.pallas_call(
        _body_dense,
        grid=(nb_tc,),
        in_specs=[pl.BlockSpec((_BR, _C), lambda i: (i + off, 0))],
        out_specs=[pl.BlockSpec((_BR, 1), lambda i: (i, 0))] * 4,
        out_shape=[stat, stat, stat, stat],
    )(pred)
    full = jax.ShapeDtypeStruct((n, 1), jnp.float32)
    sc16 = lambda: pl.BlockSpec((_NSC, _L), lambda: (0, 0))
    tc1 = lambda: pl.BlockSpec((n - _NSC, 1), lambda: (0, 0))
    rows = pl.pallas_call(
        _body_combine,
        in_specs=[
            pl.BlockSpec((n, 1), lambda: (0, 0)),
            pl.BlockSpec((n, _LW), lambda: (0, 0)),
            sc16(), sc16(), sc16(), sc16(),
            tc1(), tc1(), tc1(), tc1(),
        ],
        out_specs=pl.BlockSpec((n, 1), lambda: (0, 0)),
        out_shape=full,
    )(t2, win,
      ms.reshape(_NSC, _L), ses.reshape(_NSC, _L),
      sxs.reshape(_NSC, _L), p0s.reshape(_NSC, _L),
      mt, set_, sxt, p0t)
    return jnp.mean(rows)


# speedup vs baseline: 3.4005x; 3.4005x over previous
import functools
import jax
import jax.numpy as jnp
from jax import lax
from jax.experimental import pallas as pl
from jax.experimental.pallas import tpu as pltpu, tpu_sc as plsc

_C = 32000
_PAD = 0
_SM = 0.1
_CONF = 1.0 - _SM
_EPS = _SM / (_C - 2)
_BR = 128
_LW = 128
_NCH = _C // _LW

_info = plsc.get_sparse_core_info()
_NC, _NS, _L = _info.num_cores, _info.num_subcores, _info.num_lanes
_NW = _NC * _NS
_BATCH = 32


def _sc_gather(pred_hbm, tgt_hbm, out_hbm, t_v, rows_v, comp_v, sem):
    n_per_w = tgt_hbm.shape[0] // _NW
    wid = lax.axis_index("s") * _NC + lax.axis_index("c")
    base = wid * n_per_w
    pltpu.sync_copy(tgt_hbm.at[pl.ds(base, n_per_w)], t_v)
    for b in range(n_per_w // _BATCH):
        copies = []
        for k in range(_BATCH):
            j = b * _BATCH + k
            t = t_v[pl.ds((j // _L) * _L, _L)][j % _L]
            ct = pl.multiple_of(jnp.bitwise_and(t, -128), 128)
            r0 = pl.multiple_of(base + (j // 8) * 8, 8)
            copies.append(
                pltpu.async_copy(
                    pred_hbm.at[pl.ds(r0, 8), pl.ds(ct, 128)],
                    rows_v.at[k],
                    sem,
                )
            )
        for cp in copies:
            cp.wait()
        for k in range(_BATCH):
            j = b * _BATCH + k
            for g in range(_LW // _L):
                comp_v[pl.ds(j * _LW + g * _L, _L)] = (
                    rows_v[k, j % 8, pl.ds(g * _L, _L)])
    pltpu.sync_copy(comp_v, out_hbm.at[pl.ds(base * _LW, n_per_w * _LW)])


def _gather_windows(pred, target):
    n = target.shape[0]
    n_per_w = n // _NW
    mesh = plsc.VectorSubcoreMesh(core_axis_name="c", subcore_axis_name="s")
    f = functools.partial(
        pl.kernel,
        mesh=mesh,
        out_type=jax.ShapeDtypeStruct((n * _LW,), jnp.float32),
        scratch_types=[
            pltpu.VMEM((n_per_w,), jnp.int32),
            pltpu.VMEM((_BATCH, 8, _LW), jnp.float32),
            pltpu.VMEM((n_per_w * _LW,), jnp.float32),
            pltpu.SemaphoreType.DMA,
        ],
    )(_sc_gather)
    return f(pred, target)


_RW = 24
_NSC = _RW * _NW
_CCH = 3200
_NCC = _C // _CCH
_VPC = _CCH // _L


def _sc_dense(pred_hbm, m_hbm, se_hbm, sx_hbm, p0_hbm,
              buf, mo, seo, sxo, p0o, tb8):
    wid = lax.axis_index("s") * _NC + lax.axis_index("c")
    row0w = wid * _RW
    zer = jnp.zeros((_L,), jnp.float32)
    for g in range(_RW // 8):
        r0 = pl.multiple_of(row0w + g * 8, 8)

        def chunk(cc, carry):
            cc128 = pl.multiple_of(cc * _CCH, 128)
            pltpu.sync_copy(pred_hbm.at[pl.ds(r0, 8), pl.ds(cc128, _CCH)],
                            buf)
            out = []
            for k in range(8):
                macc, seacc, sxacc = carry[3 * k:3 * k + 3]

                def pa(v, c):
                    x = buf[k, pl.ds(v * _L, _L)]
                    return (jnp.maximum(c[0], x), c[1] + x)

                cm, csx = lax.fori_loop(
                    0, _VPC, pa,
                    (jnp.full((_L,), -jnp.inf, jnp.float32), zer),
                    unroll=8)
                mnew = jnp.maximum(macc, cm)
                scale = jnp.exp(macc - mnew)

                def pb(v, s):
                    return s + jnp.exp(buf[k, pl.ds(v * _L, _L)] - mnew)

                cse = lax.fori_loop(0, _VPC, pb, zer, unroll=8)
                out += [mnew, seacc * scale + cse, sxacc + csx]
            return tuple(out)

        ninf = jnp.full((_L,), -jnp.inf, jnp.float32)
        init = tuple([ninf, zer, zer] * 8)
        fin = lax.fori_loop(0, _NCC, chunk, init)
        for k in range(8):
            macc, seacc, sxacc = fin[3 * k:3 * k + 3]
            o16 = (g * 8 + k) * _L
            mo[pl.ds(o16, _L)] = macc
            seo[pl.ds(o16, _L)] = seacc
            sxo[pl.ds(o16, _L)] = sxacc
    for g in range(_RW // 8):
        r0 = pl.multiple_of(row0w + g * 8, 8)
        pltpu.sync_copy(pred_hbm.at[pl.ds(r0, 8), pl.ds(0, _LW)], tb8)
        for k in range(8):
            p0o[pl.ds((g * 8 + k) * _L, _L)] = tb8[k, pl.ds(0, _L)]
    pltpu.sync_copy(mo, m_hbm.at[pl.ds(row0w * _L, _RW * _L)])
    pltpu.sync_copy(seo, se_hbm.at[pl.ds(row0w * _L, _RW * _L)])
    pltpu.sync_copy(sxo, sx_hbm.at[pl.ds(row0w * _L, _RW * _L)])
    pltpu.sync_copy(p0o, p0_hbm.at[pl.ds(row0w * _L, _RW * _L)])


def _sc_dense_stats(pred):
    mesh = plsc.VectorSubcoreMesh(core_axis_name="c", subcore_axis_name="s")
    o = jax.ShapeDtypeStruct((_NSC * _L,), jnp.float32)
    f = functools.partial(
        pl.kernel,
        mesh=mesh,
        out_type=[o, o, o, o],
        scratch_types=[
            pltpu.VMEM((8, _CCH), jnp.float32),
            pltpu.VMEM((_RW * _L,), jnp.float32),
            pltpu.VMEM((_RW * _L,), jnp.float32),
            pltpu.VMEM((_RW * _L,), jnp.float32),
            pltpu.VMEM((_RW * _L,), jnp.float32),
            pltpu.VMEM((8, _LW), jnp.float32),
        ],
    )(_sc_dense)
    return f(pred)


def _body_dense(x_ref, m_ref, se_ref, sx_ref, p0_ref):
    lane = jax.lax.broadcasted_iota(jnp.int32, (_BR, _LW), 1)

    m = x_ref[:, 0:_LW]
    sp = m
    for c in range(1, _NCH):
        x = x_ref[:, c * _LW:(c + 1) * _LW]
        m = jnp.maximum(m, x)
        sp = sp + x

    mb = jnp.max(m, axis=1, keepdims=True)

    s = jnp.exp(x_ref[:, 0:_LW] - mb)
    for c in range(1, _NCH):
        s = s + jnp.exp(x_ref[:, c * _LW:(c + 1) * _LW] - mb)

    m_ref[...] = mb
    se_ref[...] = jnp.sum(s, axis=1, keepdims=True)
    sx_ref[...] = jnp.sum(sp, axis=1, keepdims=True)
    x0 = x_ref[:, 0:_LW]
    p0_ref[...] = jnp.sum(jnp.where(lane == 0, x0, 0.0), axis=1, keepdims=True)


def _body_combine(t_ref, w_ref, m16_ref, se16_ref, sx16_ref, p016_ref,
                  mt_ref, set_ref, sxt_ref, p0t_ref, o_ref):
    n = t_ref.shape[0]
    tb = t_ref[...]
    lane = jax.lax.broadcasted_iota(jnp.int32, (n, _LW), 1)
    m16 = m16_ref[...]
    mb_sc = jnp.max(m16, axis=1, keepdims=True)
    se_sc = jnp.sum(se16_ref[...] * jnp.exp(m16 - mb_sc),
                    axis=1, keepdims=True)
    z_sc = mb_sc + jnp.log(se_sc)
    sx_sc = jnp.sum(sx16_ref[...], axis=1, keepdims=True)
    p0_sc = p016_ref[:, 0:1]
    z_tc = mt_ref[...] + jnp.log(set_ref[...])
    z = jnp.concatenate([z_sc, z_tc], axis=0)
    spr = jnp.concatenate([sx_sc, sxt_ref[...]], axis=0)
    p0 = jnp.concatenate([p0_sc, p0t_ref[...]], axis=0)
    tlane = jnp.bitwise_and(tb, 127)
    ptr = jnp.sum(jnp.where(lane == tlane, w_ref[...], 0.0),
                  axis=1, keepdims=True)
    lt = ptr - z
    l0 = p0 - z
    srow = spr - _C * z
    loss = -(_CONF * lt + _EPS * (srow - l0 - lt))
    o_ref[...] = jnp.where(tb == _PAD, 0.0, loss)


def kernel(pred, target):
    n = pred.shape[0]
    nb_tc = (n - _NSC) // _BR
    off = _NSC // _BR
    t32 = target.astype(jnp.int32)
    win = _gather_windows(pred, t32).reshape(n, _LW)
    ms, ses, sxs, p0s = _sc_dense_stats(pred)
    t2 = t32.reshape(n, 1)
    stat = jax.ShapeDtypeStruct((n - _NSC, 1), jnp.float32)
    mt, set_, sxt, p0t = pl.pallas_call(
        _body_dense,
        grid=(nb_tc,),
        in_specs=[pl.BlockSpec((_BR, _C), lambda i: (i + off, 0))],
        out_specs=[pl.BlockSpec((_BR, 1), lambda i: (i, 0))] * 4,
        out_shape=[stat, stat, stat, stat],
    )(pred)
    full = jax.ShapeDtypeStruct((n, 1), jnp.float32)
    sc16 = lambda: pl.BlockSpec((_NSC, _L), lambda: (0, 0))
    tc1 = lambda: pl.BlockSpec((n - _NSC, 1), lambda: (0, 0))
    rows = pl.pallas_call(
        _body_combine,
        in_specs=[
            pl.BlockSpec((n, 1), lambda: (0, 0)),
            pl.BlockSpec((n, _LW), lambda: (0, 0)),
            sc16(), sc16(), sc16(), sc16(),
            tc1(), tc1(), tc1(), tc1(),
        ],
        out_specs=pl.BlockSpec((n, 1), lambda: (0, 0)),
        out_shape=full,
    )(t2, win,
      ms.reshape(_NSC, _L), ses.reshape(_NSC, _L),
      sxs.reshape(_NSC, _L), p0s.reshape(_NSC, _L),
      mt, set_, sxt, p0t)
    return jnp.mean(rows)


# speedup vs baseline: 4.2136x; 1.2391x over previous
import functools
import jax
import jax.numpy as jnp
from jax import lax
from jax.experimental import pallas as pl
from jax.experimental.pallas import tpu as pltpu, tpu_sc as plsc

_C = 32000
_PAD = 0
_SM = 0.1
_CONF = 1.0 - _SM
_EPS = _SM / (_C - 2)
_BR = 128
_LW = 128
_NCH = _C // _LW

_info = plsc.get_sparse_core_info()
_NC, _NS, _L = _info.num_cores, _info.num_subcores, _info.num_lanes
_NW = _NC * _NS
_BATCH = 32


def _sc_gather(pred_hbm, tgt_hbm, out_hbm, t_v, rows_v, comp_v, sem):
    n_per_w = tgt_hbm.shape[0] // _NW
    wid = lax.axis_index("s") * _NC + lax.axis_index("c")
    base = wid * n_per_w
    pltpu.sync_copy(tgt_hbm.at[pl.ds(base, n_per_w)], t_v)
    for b in range(n_per_w // _BATCH):
        copies = []
        for k in range(_BATCH):
            j = b * _BATCH + k
            t = t_v[pl.ds((j // _L) * _L, _L)][j % _L]
            ct = pl.multiple_of(jnp.bitwise_and(t, -128), 128)
            r0 = pl.multiple_of(base + (j // 8) * 8, 8)
            copies.append(
                pltpu.async_copy(
                    pred_hbm.at[pl.ds(r0, 8), pl.ds(ct, 128)],
                    rows_v.at[k],
                    sem,
                )
            )
        for cp in copies:
            cp.wait()
        for k in range(_BATCH):
            j = b * _BATCH + k
            for g in range(_LW // _L):
                comp_v[pl.ds(j * _LW + g * _L, _L)] = (
                    rows_v[k, j % 8, pl.ds(g * _L, _L)])
    pltpu.sync_copy(comp_v, out_hbm.at[pl.ds(base * _LW, n_per_w * _LW)])


def _gather_windows(pred, target):
    n = target.shape[0]
    n_per_w = n // _NW
    mesh = plsc.VectorSubcoreMesh(core_axis_name="c", subcore_axis_name="s")
    f = functools.partial(
        pl.kernel,
        mesh=mesh,
        out_type=jax.ShapeDtypeStruct((n * _LW,), jnp.float32),
        scratch_types=[
            pltpu.VMEM((n_per_w,), jnp.int32),
            pltpu.VMEM((_BATCH, 8, _LW), jnp.float32),
            pltpu.VMEM((n_per_w * _LW,), jnp.float32),
            pltpu.SemaphoreType.DMA,
        ],
    )(_sc_gather)
    return f(pred, target)


def _body_dense(x_ref, z_ref, sp_ref, p0_ref):
    lane = jax.lax.broadcasted_iota(jnp.int32, (_BR, _LW), 1)

    m = x_ref[:, 0:_LW]
    sp = m
    for c in range(1, _NCH):
        x = x_ref[:, c * _LW:(c + 1) * _LW]
        m = jnp.maximum(m, x)
        sp = sp + x

    mb = jnp.max(m, axis=1, keepdims=True)

    s = jnp.exp(x_ref[:, 0:_LW] - mb)
    for c in range(1, _NCH):
        s = s + jnp.exp(x_ref[:, c * _LW:(c + 1) * _LW] - mb)

    z_ref[...] = mb + jnp.log(jnp.sum(s, axis=1, keepdims=True))
    sp_ref[...] = jnp.sum(sp, axis=1, keepdims=True)
    x0 = x_ref[:, 0:_LW]
    p0_ref[...] = jnp.sum(jnp.where(lane == 0, x0, 0.0), axis=1, keepdims=True)


def _body_combine(t_ref, w_ref, z_ref, sp_ref, p0_ref, o_ref):
    n = t_ref.shape[0]
    tb = t_ref[...]
    lane = jax.lax.broadcasted_iota(jnp.int32, (n, _LW), 1)
    z = z_ref[...]
    spr = sp_ref[...]
    p0 = p0_ref[...]
    tlane = jnp.bitwise_and(tb, 127)
    ptr = jnp.sum(jnp.where(lane == tlane, w_ref[...], 0.0),
                  axis=1, keepdims=True)
    lt = ptr - z
    l0 = p0 - z
    srow = spr - _C * z
    loss = -(_CONF * lt + _EPS * (srow - l0 - lt))
    o_ref[...] = jnp.where(tb == _PAD, 0.0, loss)


def kernel(pred, target):
    n = pred.shape[0]
    nb = n // _BR
    t32 = target.astype(jnp.int32)
    win = _gather_windows(pred, t32).reshape(n, _LW)
    t2 = t32.reshape(n, 1)
    stat = jax.ShapeDtypeStruct((n, 1), jnp.float32)
    z, sp, p0 = pl.pallas_call(
        _body_dense,
        grid=(nb,),
        in_specs=[pl.BlockSpec((_BR, _C), lambda i: (i, 0))],
        out_specs=[pl.BlockSpec((_BR, 1), lambda i: (i, 0))] * 3,
        out_shape=[stat, stat, stat],
    )(pred)
    rows = pl.pallas_call(
        _body_combine,
        in_specs=[
            pl.BlockSpec((n, 1), lambda: (0, 0)),
            pl.BlockSpec((n, _LW), lambda: (0, 0)),
            pl.BlockSpec((n, 1), lambda: (0, 0)),
            pl.BlockSpec((n, 1), lambda: (0, 0)),
            pl.BlockSpec((n, 1), lambda: (0, 0)),
        ],
        out_specs=pl.BlockSpec((n, 1), lambda: (0, 0)),
        out_shape=stat,
    )(t2, win, z, sp, p0)
    return jnp.mean(rows)
